# Initial kernel scaffold; baseline (speedup 1.0000x reference)
import jax
import jax.numpy as jnp
from jax.experimental import pallas as pl


def kernel(L_init_W, L_init_b, C_init_W, C_init_b, L_msg_pos, L_msg_neg, C_msg_pos, C_msg_neg, L_update, C_update, var_vote, var_idx_pos, cls_idx_pos, var_idx_neg, cls_idx_neg):
    raise NotImplementedError("write your pallas kernel here")


# speedup vs baseline: 1.7775x; 1.7775x over previous
import functools

import jax
import jax.numpy as jnp
from jax import lax
from jax.experimental import pallas as pl
from jax.experimental.pallas import tpu as pltpu
from jax.experimental.pallas import tpu_sc as plsc

NV = 10000
NCL = 40000
D = 128
ROUNDS = 8
E = 160000

NC = 2
NS = 16
KB = 128
NB = 80
EP = NS * NB * KB

CH = 10240
NCHUNK = 4
NCL_P = CH * NCHUNK
ACC_A = CH + KB
ACC_B = 10240

_f32 = jnp.float32
_i32 = jnp.int32
_mesh = plsc.VectorSubcoreMesh(core_axis_name="c", subcore_axis_name="s")


@functools.partial(
    pl.kernel,
    out_type=jax.ShapeDtypeStruct((NC, NCL_P, D), _f32),
    mesh=_mesh,
    scratch_types=[
        pltpu.VMEM((NB, KB), _i32),
        pltpu.VMEM((NB, KB), _i32),
        pltpu.VMEM((KB, D), _f32),
        pltpu.VMEM((2, 32), _i32),
        pltpu.VMEM_SHARED((ACC_A, D), _f32),
    ],
)
def _sc_l2c(table, gidx, dst4, bnds, out, idxg_v, idxd_v, rows_v, bnd_v, acc):
    c = lax.axis_index("c")
    s = lax.axis_index("s")

    pltpu.sync_copy(gidx.at[c, s], idxg_v)
    for q in range(NCHUNK):
        @pl.loop(0, KB)
        def _(i):
            for t in range(D // 16):
                rows_v[i, pl.ds(16 * t, 16)] = jnp.zeros((16,), _f32)

        stripe = ACC_A // NS
        @pl.loop(0, stripe // KB)
        def _(j):
            pltpu.sync_copy(rows_v, acc.at[pl.ds(s * stripe + j * KB, KB)])
        rem = stripe % KB
        pltpu.sync_copy(rows_v.at[pl.ds(0, rem)],
                        acc.at[pl.ds(s * stripe + stripe - rem, rem)])
        pltpu.sync_copy(dst4.at[c, q, s], idxd_v)
        pltpu.sync_copy(bnds.at[c, q], bnd_v)
        plsc.subcore_barrier()

        blo = bnd_v[0, pl.ds(s, 16)][0]
        bhi = bnd_v[1, pl.ds(s, 16)][0]

        @pl.loop(0, NB)
        def _(b):
            @pl.when(jnp.logical_and(b >= blo, b < bhi))
            def _():
                pltpu.sync_copy(table.at[idxg_v.at[b]], rows_v)
                pltpu.sync_copy(rows_v, acc.at[idxd_v.at[b]], add=True)

        plsc.subcore_barrier()
        rw = CH // NS
        pltpu.sync_copy(acc.at[pl.ds(s * rw, rw)],
                        out.at[c, pl.ds(q * CH + s * rw, rw)])
        plsc.subcore_barrier()


@functools.partial(
    pl.kernel,
    out_type=jax.ShapeDtypeStruct((NC, ACC_B, D), _f32),
    mesh=_mesh,
    scratch_types=[
        pltpu.VMEM((NB, KB), _i32),
        pltpu.VMEM((NB, KB), _i32),
        pltpu.VMEM((KB, D), _f32),
        pltpu.VMEM_SHARED((ACC_B, D), _f32),
    ],
)
def _sc_c2l(table, gidx, dst, out, idxg_v, idxd_v, rows_v, acc):
    c = lax.axis_index("c")
    s = lax.axis_index("s")

    @pl.loop(0, KB)
    def _(i):
        for t in range(D // 16):
            rows_v[i, pl.ds(16 * t, 16)] = jnp.zeros((16,), _f32)

    stripe = ACC_B // NS
    @pl.loop(0, stripe // KB)
    def _(j):
        pltpu.sync_copy(rows_v, acc.at[pl.ds(s * stripe + j * KB, KB)])
    plsc.subcore_barrier()
    pltpu.sync_copy(gidx.at[c, s], idxg_v)
    pltpu.sync_copy(dst.at[c, s], idxd_v)

    @pl.loop(0, NB)
    def _(b):
        pltpu.sync_copy(table.at[idxg_v.at[b]], rows_v)
        pltpu.sync_copy(rows_v, acc.at[idxd_v.at[b]], add=True)

    plsc.subcore_barrier()
    pltpu.sync_copy(acc.at[pl.ds(s * stripe, stripe)],
                    out.at[c, pl.ds(s * stripe, stripe)])


def _mm(a, b):
    return jnp.dot(a, b, preferred_element_type=_f32)


def _msg_body(h_ref, w1, b1, w2, b2, w3, b3, out_ref):
    h = h_ref[...]
    for p in range(2):
        x = jnp.maximum(_mm(h, w1[p]) + b1[p], 0.0)
        x = jnp.maximum(_mm(x, w2[p]) + b2[p], 0.0)
        out_ref[p] = _mm(x, w3[p]) + b3[p]


def _msg_call(h, mlp, n, blk):
    w1, b1, w2, b2, w3, b3 = mlp
    wspec = lambda shp: pl.BlockSpec(shp, lambda i: (0,) * len(shp))
    return pl.pallas_call(
        _msg_body,
        grid=(n // blk,),
        in_specs=[
            pl.BlockSpec((blk, D), lambda i: (i, 0)),
            wspec((2, D, D)), wspec((2, 1, D)),
            wspec((2, D, D)), wspec((2, 1, D)),
            wspec((2, D, D)), wspec((2, 1, D)),
        ],
        out_specs=pl.BlockSpec((2, blk, D), lambda i: (0, i, 0)),
        out_shape=jax.ShapeDtypeStruct((2, n, D), _f32),
    )(h, w1, b1, w2, b2, w3, b3)


def _lstm_msg_body(n_msg, m_ref, h_ref, c_ref, wih, whh, bias,
                   w1, b1, w2, b2, w3, b3, h_out, c_out, msg_out):
    m = m_ref[0] + m_ref[1]
    g = _mm(m, wih[...]) + _mm(h_ref[...], whh[...]) + bias[...]
    ii = jax.nn.sigmoid(g[:, 0:D])
    ff = jax.nn.sigmoid(g[:, D:2 * D])
    gg = jnp.tanh(g[:, 2 * D:3 * D])
    oo = jax.nn.sigmoid(g[:, 3 * D:4 * D])
    c2 = ff * c_ref[...] + ii * gg
    h2 = oo * jnp.tanh(c2)
    h_out[...] = h2
    c_out[...] = c2
    for p in range(n_msg):
        x = jnp.maximum(_mm(h2, w1[p]) + b1[p], 0.0)
        x = jnp.maximum(_mm(x, w2[p]) + b2[p], 0.0)
        msg_out[p] = _mm(x, w3[p]) + b3[p]


def _lstm_msg_call(msum, h, c, lstm, mlp, n, blk, n_msg):
    wih, whh, bias = lstm
    w1, b1, w2, b2, w3, b3 = mlp
    wspec = lambda shp: pl.BlockSpec(shp, lambda i: (0,) * len(shp))
    return pl.pallas_call(
        functools.partial(_lstm_msg_body, n_msg),
        grid=(n // blk,),
        in_specs=[
            pl.BlockSpec((2, blk, D), lambda i: (0, i, 0)),
            pl.BlockSpec((blk, D), lambda i: (i, 0)),
            pl.BlockSpec((blk, D), lambda i: (i, 0)),
            wspec((D, 4 * D)), wspec((D, 4 * D)), wspec((1, 4 * D)),
            wspec((n_msg, D, D)), wspec((n_msg, 1, D)),
            wspec((n_msg, D, D)), wspec((n_msg, 1, D)),
            wspec((n_msg, D, D)), wspec((n_msg, 1, D)),
        ],
        out_specs=[
            pl.BlockSpec((blk, D), lambda i: (i, 0)),
            pl.BlockSpec((blk, D), lambda i: (i, 0)),
            pl.BlockSpec((n_msg, blk, D), lambda i: (0, i, 0)),
        ],
        out_shape=[
            jax.ShapeDtypeStruct((n, D), _f32),
            jax.ShapeDtypeStruct((n, D), _f32),
            jax.ShapeDtypeStruct((n_msg, n, D), _f32),
        ],
    )(msum, h, c, wih, whh, bias, w1, b1, w2, b2, w3, b3)


def _pad_e(idx, pad_val):
    idx = idx.astype(_i32)
    return jnp.concatenate([idx, jnp.full((EP - E,), pad_val, _i32)])


def _stack_mlp(p_pos, p_neg):
    outs = []
    for a, b in zip(p_pos, p_neg):
        if a.ndim == 2:
            outs.append(jnp.stack([a.T, b.T]))
        else:
            outs.append(jnp.stack([a.reshape(1, -1), b.reshape(1, -1)]))
    return tuple(outs)


def _prep_l2c(var_idx, cls_idx):
    cls_p = _pad_e(cls_idx, NCL_P - 1)
    var_p = _pad_e(var_idx, 0)
    order = jnp.argsort(cls_p)
    cls_s = cls_p[order]
    var_s = var_p[order]
    chunk_of = cls_s // CH
    dst4 = jnp.stack([
        jnp.where(chunk_of == q, cls_s - q * CH, CH) for q in range(NCHUNK)
    ]).reshape(NCHUNK, NS, NB, KB)
    starts = jnp.searchsorted(
        cls_s, jnp.arange(NCHUNK, dtype=_i32) * CH).astype(_i32)
    ends = jnp.concatenate([starts[1:], jnp.array([EP], _i32)])
    tile_base = (jnp.arange(NS, dtype=_i32) * NB)[None, :]
    blo = jnp.clip(starts[:, None] // KB - tile_base, 0, NB)
    bhi = jnp.clip((ends[:, None] + KB - 1) // KB - tile_base, 0, NB)
    bnds = jnp.stack([blo, bhi], axis=1)
    bnds = jnp.pad(bnds, ((0, 0), (0, 0), (0, 16)))
    return var_s.reshape(NS, NB, KB), dst4, bnds


def kernel(L_init_W, L_init_b, C_init_W, C_init_b, L_msg_pos, L_msg_neg,
           C_msg_pos, C_msg_neg, L_update, C_update, var_vote,
           var_idx_pos, cls_idx_pos, var_idx_neg, cls_idx_neg):
    gp, dst4p, bndp = _prep_l2c(var_idx_pos, cls_idx_pos)
    gn, dst4n, bndn = _prep_l2c(var_idx_neg, cls_idx_neg)
    gidx_a = jnp.stack([gp, gn + NV])
    dst_a = jnp.stack([dst4p, dst4n])
    bnds_a = jnp.stack([bndp, bndn])

    cshape = (NS, NB, KB)
    gidx_b = jnp.stack([_pad_e(cls_idx_pos, 0).reshape(cshape),
                        _pad_e(cls_idx_neg, 0).reshape(cshape) + NCL_P])
    dst_b = jnp.stack([_pad_e(var_idx_pos, NV).reshape(cshape),
                       _pad_e(var_idx_neg, NV).reshape(cshape)])

    l_mlp = _stack_mlp(L_msg_pos, L_msg_neg)
    c_mlp = _stack_mlp(C_msg_pos, C_msg_neg)
    vw1, vb1, vw2, vb2, vw3, vb3 = var_vote
    vw3p = jnp.zeros((D, D), _f32).at[:, :1].set(vw3.T)
    vb3p = jnp.zeros((1, D), _f32).at[:, :1].set(vb3.reshape(1, 1))
    vote_mlp = (vw1.T[None], vb1.reshape(1, 1, D), vw2.T[None],
                vb2.reshape(1, 1, D), vw3p[None], vb3p[None])

    def lstm_prep(p):
        wih, whh, bih, bhh = p
        return (wih.T, whh.T, (bih + bhh).reshape(1, 4 * D))

    l_lstm = lstm_prep(L_update)
    c_lstm = lstm_prep(C_update)

    l_row = (L_init_W[:, 0] + L_init_b).reshape(1, D)
    c_row = (C_init_W[:, 0] + C_init_b).reshape(1, D)
    L_h = jnp.tile(l_row, (NV, 1))
    C_h = jnp.tile(c_row, (NCL_P, 1))
    L_c = jnp.zeros((NV, D), _f32)
    C_c = jnp.zeros((NCL_P, D), _f32)

    lpln = _msg_call(L_h, l_mlp, NV, 400)
    for r in range(ROUNDS):
        lc = _sc_l2c(lpln.reshape(2 * NV, D), gidx_a, dst_a, bnds_a)
        C_h, C_c, cpcn = _lstm_msg_call(lc, C_h, C_c, c_lstm, c_mlp,
                                        NCL_P, 640, 2)
        cl = _sc_c2l(cpcn.reshape(2 * NCL_P, D), gidx_b, dst_b)
        if r < ROUNDS - 1:
            L_h, L_c, lpln = _lstm_msg_call(cl, L_h, L_c, l_lstm, l_mlp,
                                            NV, 400, 2)
        else:
            L_h, L_c, vote = _lstm_msg_call(cl, L_h, L_c, l_lstm, vote_mlp,
                                            NV, 400, 1)
    return vote[0, :, :1]
